# baseline (speedup 1.0000x reference)
import jax
import jax.numpy as jnp
from jax import lax
from jax.experimental import pallas as pl
from jax.experimental.pallas import tpu as pltpu
from jax.experimental.pallas import tpu_sc as plsc

_HIDDEN = 2048
_EXPERTS = 64
_TOKENS = 8192
_BT = 2048

_LANES = 16
_NC = 2
_NS = 16
_NW = _NC * _NS
_TPW = _TOKENS // _NW
_GROUPS = _TPW // _LANES


def _gate_matmul_body(x_ref, w_ref, out_ref, outT_ref):
    res = lax.dot_general(
        x_ref[...], w_ref[...],
        dimension_numbers=(((1,), (1,)), ((), ())),
        preferred_element_type=jnp.float32,
    )
    out_ref[...] = res
    outT_ref[...] = res.T


def _gate_matmul(x, W):
    return pl.pallas_call(
        _gate_matmul_body,
        grid=(_TOKENS // _BT,),
        in_specs=[
            pl.BlockSpec((_BT, _HIDDEN), lambda i: (i, 0)),
            pl.BlockSpec((_EXPERTS, _HIDDEN), lambda i: (0, 0)),
        ],
        out_specs=[
            pl.BlockSpec((_BT, _EXPERTS), lambda i: (i, 0)),
            pl.BlockSpec((_EXPERTS, _BT), lambda i: (0, i)),
        ],
        out_shape=[
            jax.ShapeDtypeStruct((_TOKENS, _EXPERTS), jnp.float32),
            jax.ShapeDtypeStruct((_EXPERTS, _TOKENS), jnp.float32),
        ],
    )(x, W)


def _route_body(logitsT_hbm, idx_hbm, prob_hbm, slab_v, idx_v, prob_v):
    wid = lax.axis_index("s") * _NC + lax.axis_index("c")
    base = wid * _TPW
    pltpu.sync_copy(logitsT_hbm.at[:, pl.ds(base, _TPW)], slab_v)

    lane_iota = lax.iota(jnp.int32, _LANES)
    neg_inf = jnp.full((_LANES,), -jnp.inf, jnp.float32)
    zeros = jnp.zeros((_LANES,), jnp.int32)
    ones = jnp.full((_LANES,), 1, jnp.int32)

    unroll = 4

    def per_block(b, _):
        t_ids = [(b * unroll + u) * _LANES + lane_iota for u in range(unroll)]
        t_off = [(b * unroll + u) * _LANES for u in range(unroll)]
        m1 = [neg_inf] * unroll
        a1 = [zeros] * unroll
        m2 = [neg_inf] * unroll
        a2 = [zeros] * unroll
        for e in range(_EXPERTS):
            e_vec = jnp.full((_LANES,), e, jnp.int32)
            for u in range(unroll):
                v = slab_v[e, pl.ds(t_off[u], _LANES)]
                gt1 = v > m1[u]
                gt2 = v > m2[u]
                m2[u], a2[u] = (
                    jnp.where(gt1, m1[u], jnp.where(gt2, v, m2[u])),
                    jnp.where(gt1, a1[u], jnp.where(gt2, e_vec, a2[u])))
                m1[u] = jnp.where(gt1, v, m1[u])
                a1[u] = jnp.where(gt1, e_vec, a1[u])

        for u in range(unroll):
            ex = jnp.exp(m2[u] - m1[u])
            denom = ex + jnp.float32(1.0)
            p1 = jnp.float32(1.0) / denom
            p2 = ex / denom

            plsc.store_scatter(idx_v, [t_ids[u], zeros], a1[u])
            plsc.store_scatter(idx_v, [t_ids[u], ones], a2[u])
            plsc.store_scatter(prob_v, [t_ids[u], zeros], p1)
            plsc.store_scatter(prob_v, [t_ids[u], ones], p2)
        return 0

    lax.fori_loop(0, _GROUPS // unroll, per_block, 0)

    pltpu.sync_copy(idx_v, idx_hbm.at[pl.ds(base, _TPW), :])
    pltpu.sync_copy(prob_v, prob_hbm.at[pl.ds(base, _TPW), :])


def kernel(x, W):
    logits, logitsT = _gate_matmul(x, W)
    route = pl.kernel(
        _route_body,
        mesh=plsc.VectorSubcoreMesh(core_axis_name="c", subcore_axis_name="s"),
        out_type=(
            jax.ShapeDtypeStruct((_TOKENS, 2), jnp.int32),
            jax.ShapeDtypeStruct((_TOKENS, 2), jnp.float32),
        ),
        scratch_types=[
            pltpu.VMEM((_EXPERTS, _TPW), jnp.float32),
            pltpu.VMEM((_TPW, 2), jnp.int32),
            pltpu.VMEM((_TPW, 2), jnp.float32),
        ],
        compiler_params=pltpu.CompilerParams(needs_layout_passes=False),
    )
    top_k_indices, probs = route(logitsT)
    return (logits, top_k_indices, probs)


# speedup vs baseline: 1.0435x; 1.0435x over previous
import jax
import jax.numpy as jnp
from jax import lax
from jax.experimental import pallas as pl
from jax.experimental.pallas import tpu as pltpu
from jax.experimental.pallas import tpu_sc as plsc

_HIDDEN = 2048
_EXPERTS = 64
_TOKENS = 8192
_BT = 1024

_LANES = 16
_NC = 2
_NS = 16
_NW = _NC * _NS
_TPW = _TOKENS // _NW
_GROUPS = _TPW // _LANES


def _gate_matmul_body(x_ref, w_ref, out_ref, outT_ref):
    res = lax.dot_general(
        x_ref[...], w_ref[...],
        dimension_numbers=(((1,), (1,)), ((), ())),
        preferred_element_type=jnp.float32,
    )
    out_ref[...] = res
    outT_ref[...] = res.T


def _gate_matmul(x, W):
    return pl.pallas_call(
        _gate_matmul_body,
        grid=(_TOKENS // _BT,),
        in_specs=[
            pl.BlockSpec((_BT, _HIDDEN), lambda i: (i, 0)),
            pl.BlockSpec((_EXPERTS, _HIDDEN), lambda i: (0, 0)),
        ],
        out_specs=[
            pl.BlockSpec((_BT, _EXPERTS), lambda i: (i, 0)),
            pl.BlockSpec((_EXPERTS, _BT), lambda i: (0, i)),
        ],
        out_shape=[
            jax.ShapeDtypeStruct((_TOKENS, _EXPERTS), jnp.float32),
            jax.ShapeDtypeStruct((_EXPERTS, _TOKENS), jnp.float32),
        ],
    )(x, W)


def _route_body(logitsT_hbm, idx_hbm, prob_hbm, slab_v, idx_v, prob_v):
    wid = lax.axis_index("s") * _NC + lax.axis_index("c")
    base = wid * _TPW
    pltpu.sync_copy(logitsT_hbm.at[:, pl.ds(base, _TPW)], slab_v)

    lane_iota = lax.iota(jnp.int32, _LANES)
    neg_inf = jnp.full((_LANES,), -jnp.inf, jnp.float32)
    zeros = jnp.zeros((_LANES,), jnp.int32)
    ones = jnp.full((_LANES,), 1, jnp.int32)

    unroll = 4

    def per_block(b, _):
        t_ids = [(b * unroll + u) * _LANES + lane_iota for u in range(unroll)]
        t_off = [(b * unroll + u) * _LANES for u in range(unroll)]
        m1 = [neg_inf] * unroll
        a1 = [zeros] * unroll
        m2 = [neg_inf] * unroll
        a2 = [zeros] * unroll
        for e in range(_EXPERTS):
            e_vec = jnp.full((_LANES,), e, jnp.int32)
            for u in range(unroll):
                v = slab_v[e, pl.ds(t_off[u], _LANES)]
                gt1 = v > m1[u]
                gt2 = v > m2[u]
                m2[u], a2[u] = (
                    jnp.where(gt1, m1[u], jnp.where(gt2, v, m2[u])),
                    jnp.where(gt1, a1[u], jnp.where(gt2, e_vec, a2[u])))
                m1[u] = jnp.where(gt1, v, m1[u])
                a1[u] = jnp.where(gt1, e_vec, a1[u])

        for u in range(unroll):
            ex = jnp.exp(m2[u] - m1[u])
            denom = ex + jnp.float32(1.0)
            p1 = jnp.float32(1.0) / denom
            p2 = ex / denom

            plsc.store_scatter(idx_v, [t_ids[u], zeros], a1[u])
            plsc.store_scatter(idx_v, [t_ids[u], ones], a2[u])
            plsc.store_scatter(prob_v, [t_ids[u], zeros], p1)
            plsc.store_scatter(prob_v, [t_ids[u], ones], p2)
        return 0

    lax.fori_loop(0, _GROUPS // unroll, per_block, 0)

    pltpu.sync_copy(idx_v, idx_hbm.at[pl.ds(base, _TPW), :])
    pltpu.sync_copy(prob_v, prob_hbm.at[pl.ds(base, _TPW), :])


def kernel(x, W):
    logits, logitsT = _gate_matmul(x, W)
    route = pl.kernel(
        _route_body,
        mesh=plsc.VectorSubcoreMesh(core_axis_name="c", subcore_axis_name="s"),
        out_type=(
            jax.ShapeDtypeStruct((_TOKENS, 2), jnp.int32),
            jax.ShapeDtypeStruct((_TOKENS, 2), jnp.float32),
        ),
        scratch_types=[
            pltpu.VMEM((_EXPERTS, _TPW), jnp.float32),
            pltpu.VMEM((_TPW, 2), jnp.int32),
            pltpu.VMEM((_TPW, 2), jnp.float32),
        ],
        compiler_params=pltpu.CompilerParams(needs_layout_passes=False),
    )
    top_k_indices, probs = route(logitsT)
    return (logits, top_k_indices, probs)
